# baseline (speedup 1.0000x reference)
import jax
import jax.numpy as jnp
from jax import lax
from jax.experimental import pallas as pl
from jax.experimental.pallas import tpu as pltpu
from jax.experimental.pallas import tpu_sc as plsc

NUM_EXPERTS = 8
TOP_K = 2
D_MODEL = 768
D_FF = 768
M_TOKENS = 2048

NPAIRS = M_TOKENS * TOP_K
BM = 512
NB = NPAIRS // BM + NUM_EXPERTS - 1
CH = 512
NCH = NPAIRS // CH
W = 128
BT = 512


def _route_body(ids_ref, dest_ref, be_ref, nv_ref):
    ids = ids_ref[...]
    e_row = lax.broadcasted_iota(jnp.int32, (1, NUM_EXPERTS), 1)
    ri = lax.broadcasted_iota(jnp.int32, (CH, CH), 0)
    ci = lax.broadcasted_iota(jnp.int32, (CH, CH), 1)
    ltri = jnp.where(ci < ri, 1.0, 0.0).astype(jnp.bfloat16)

    run = jnp.zeros((1, NUM_EXPERTS), jnp.float32)
    ranks, ohs = [], []
    for c in range(NCH):
        slot = c // (M_TOKENS // CH)
        lo = (c % (M_TOKENS // CH)) * CH
        idc = ids[lo:lo + CH, slot:slot + 1]
        oh_b = idc == e_row
        oh = oh_b.astype(jnp.bfloat16)
        r = lax.dot_general(ltri, oh, (((1,), (0,)), ((), ())),
                            preferred_element_type=jnp.float32)
        ranks.append(r + run)
        ohs.append(oh_b)
        run = run + jnp.sum(oh.astype(jnp.float32), axis=0, keepdims=True)

    counts = run
    nb = jnp.ceil(counts / BM)
    fr = lax.broadcasted_iota(jnp.int32, (NUM_EXPERTS, NUM_EXPERTS), 0)
    er = lax.broadcasted_iota(jnp.int32, (NUM_EXPERTS, NUM_EXPERTS), 1)
    pre = jnp.where(fr < er, 1.0, 0.0).astype(jnp.bfloat16)
    pstart = lax.dot_general(nb.astype(jnp.bfloat16), pre,
                             (((1,), (0,)), ((), ())),
                             preferred_element_type=jnp.float32)
    pstart_rows = pstart * BM

    for c in range(NCH):
        dvals = pstart_rows + ranks[c]
        d = jnp.sum(jnp.where(ohs[c], dvals, 0.0), axis=1).astype(jnp.int32)
        dest_ref[c, :] = d

    b_col = lax.broadcasted_iota(jnp.int32, (NB, 1), 0).astype(jnp.float32)
    be = jnp.sum((pstart <= b_col).astype(jnp.int32), axis=1) - 1
    be_ref[0, :] = be
    nv_ref[...] = jnp.sum(nb).astype(jnp.int32).reshape(1, 1)


def _route(ids):
    return pl.pallas_call(
        _route_body,
        in_specs=[pl.BlockSpec((M_TOKENS, TOP_K), lambda: (0, 0))],
        out_specs=[
            pl.BlockSpec((NCH, CH), lambda: (0, 0)),
            pl.BlockSpec((1, NB), lambda: (0, 0)),
            pl.BlockSpec((1, 1), lambda: (0, 0)),
        ],
        out_shape=[
            jax.ShapeDtypeStruct((NCH, CH), jnp.int32),
            jax.ShapeDtypeStruct((1, NB), jnp.int32),
            jax.ShapeDtypeStruct((1, 1), jnp.int32),
        ],
    )(ids)


def _sc_mesh():
    return plsc.VectorSubcoreMesh(core_axis_name="c", subcore_axis_name="s")


DH = D_MODEL // 2


def _as_f32_rows(a16):
    n = a16.shape[0]
    return lax.bitcast_convert_type(a16.reshape(n, DH, 2), jnp.float32)


def _as_bf16_rows(a32):
    n = a32.shape[0]
    return lax.bitcast_convert_type(a32, jnp.bfloat16).reshape(n, D_MODEL)


def _dispatch_sc(x32v, dest):
    @pl.kernel(out_type=jax.ShapeDtypeStruct((NB * BM, DH), jnp.float32),
               mesh=_sc_mesh())
    def dispatch_kernel(x_hbm, d_hbm, xs_hbm):
        def body(x_vmem, d_vmem):
            pltpu.sync_copy(x_vmem, xs_hbm.at[d_vmem.at[0]])

        pltpu.emit_pipeline(
            body,
            grid=(NPAIRS // W,),
            in_specs=[
                pl.BlockSpec((W, DH),
                             index_map=lambda i: (i % (M_TOKENS // W), 0)),
                pl.BlockSpec((1, W), index_map=lambda i: (0, i)),
            ],
            out_specs=[],
            core_axis_name=("c", "s"),
            dimension_semantics=(pltpu.PARALLEL,),
        )(x_hbm, d_hbm)

    return dispatch_kernel(x32v, dest)


def _gather_sc(ys32v, dest):
    @pl.kernel(out_type=jax.ShapeDtypeStruct((NPAIRS, DH), jnp.float32),
               mesh=_sc_mesh())
    def gather_kernel(ys_hbm, d_hbm, yg_hbm):
        def body(d_vmem, o_vmem):
            pltpu.sync_copy(ys_hbm.at[d_vmem.at[0]], o_vmem)

        pltpu.emit_pipeline(
            body,
            grid=(NPAIRS // W,),
            in_specs=[pl.BlockSpec((1, W), index_map=lambda i: (0, i))],
            out_specs=[pl.BlockSpec((W, DH), index_map=lambda i: (i, 0))],
            core_axis_name=("c", "s"),
            dimension_semantics=(pltpu.PARALLEL,),
        )(d_hbm, yg_hbm)

    return gather_kernel(ys32v, dest)


def _mm_body(be_ref, nv_ref, xs_ref, w1_ref, w2_ref, ys_ref):
    b = pl.program_id(0)

    @pl.when(b < nv_ref[0])
    def _():
        x = xs_ref[...]
        h = lax.dot_general(x, w1_ref[0], (((1,), (1,)), ((), ())),
                            preferred_element_type=jnp.float32)
        gate = h[:, :D_FF]
        up = h[:, D_FF:]
        act = (jax.nn.sigmoid(gate) * gate * up).astype(jnp.bfloat16)
        y = lax.dot_general(act, w2_ref[0], (((1,), (1,)), ((), ())),
                            preferred_element_type=jnp.float32)
        ys_ref[...] = y.astype(jnp.bfloat16)


def _grouped_mlp(be, nv, xs, w1_16, w2_16):
    def clamp(b, nv_ref):
        return jnp.minimum(b, nv_ref[0] - 1)

    grid_spec = pltpu.PrefetchScalarGridSpec(
        num_scalar_prefetch=2,
        grid=(NB,),
        in_specs=[
            pl.BlockSpec((BM, D_MODEL),
                         lambda b, be_r, nv_r: (clamp(b, nv_r), 0)),
            pl.BlockSpec((1, 2 * D_FF, D_MODEL),
                         lambda b, be_r, nv_r: (be_r[clamp(b, nv_r)], 0, 0)),
            pl.BlockSpec((1, D_MODEL, D_FF),
                         lambda b, be_r, nv_r: (be_r[clamp(b, nv_r)], 0, 0)),
        ],
        out_specs=pl.BlockSpec((BM, D_MODEL),
                               lambda b, be_r, nv_r: (clamp(b, nv_r), 0)),
    )
    return pl.pallas_call(
        _mm_body,
        grid_spec=grid_spec,
        out_shape=jax.ShapeDtypeStruct((NB * BM, D_MODEL), jnp.bfloat16),
    )(be, nv, xs, w1_16, w2_16)


def _fin_body(tw_ref, y0_ref, y1_ref, out_ref):
    tw = tw_ref[...]
    y0 = y0_ref[...].astype(jnp.float32)
    y1 = y1_ref[...].astype(jnp.float32)
    out_ref[...] = y0 * tw[:, 0:1] + y1 * tw[:, 1:2]


def _combine(topk_weights, yg):
    return pl.pallas_call(
        _fin_body,
        grid=(M_TOKENS // BT,),
        in_specs=[
            pl.BlockSpec((BT, TOP_K), lambda m: (m, 0)),
            pl.BlockSpec((BT, D_MODEL), lambda m: (m, 0)),
            pl.BlockSpec((BT, D_MODEL), lambda m: (m + M_TOKENS // BT, 0)),
        ],
        out_specs=pl.BlockSpec((BT, D_MODEL), lambda m: (m, 0)),
        out_shape=jax.ShapeDtypeStruct((M_TOKENS, D_MODEL), jnp.float32),
    )(topk_weights, yg, yg)


def kernel(hidden_states, w1, w2, topk_weights, topk_ids):
    x16 = hidden_states.astype(jnp.bfloat16)
    w1_16 = w1.astype(jnp.bfloat16)
    w2_16 = w2.astype(jnp.bfloat16)
    ids = topk_ids.astype(jnp.int32)

    dest8, be2, nv2 = _route(ids)
    dest = dest8.reshape(1, NPAIRS)
    be = be2.reshape(NB)
    nv = nv2.reshape(1)

    xs = _as_bf16_rows(_dispatch_sc(_as_f32_rows(x16), dest))
    ys = _grouped_mlp(be, nv, xs, w1_16, w2_16)
    yg = _as_bf16_rows(_gather_sc(_as_f32_rows(ys), dest))
    return _combine(topk_weights, yg)


# speedup vs baseline: 5.5128x; 5.5128x over previous
import jax
import jax.numpy as jnp
from jax import lax
from jax.experimental import pallas as pl
from jax.experimental.pallas import tpu as pltpu
from jax.experimental.pallas import tpu_sc as plsc

NUM_EXPERTS = 8
TOP_K = 2
D_MODEL = 768
D_FF = 768
M_TOKENS = 2048

NPAIRS = M_TOKENS * TOP_K
BM = 512
NB = NPAIRS // BM + NUM_EXPERTS - 1
NROWS = NB * BM
CH = 512
NCH = NPAIRS // CH
W = 128
DH = D_MODEL // 2
BT = 512


def _route_body(ids_ref, dest_ref, be_ref, nv_ref):
    ids = ids_ref[...]
    e_row = lax.broadcasted_iota(jnp.int32, (1, NUM_EXPERTS), 1)
    ri = lax.broadcasted_iota(jnp.int32, (CH, CH), 0)
    ci = lax.broadcasted_iota(jnp.int32, (CH, CH), 1)
    ltri = jnp.where(ci < ri, 1.0, 0.0).astype(jnp.bfloat16)

    run = jnp.zeros((1, NUM_EXPERTS), jnp.float32)
    ranks, ohs = [], []
    for c in range(NCH):
        slot = c // (M_TOKENS // CH)
        lo = (c % (M_TOKENS // CH)) * CH
        idc = ids[lo:lo + CH, slot:slot + 1]
        oh_b = idc == e_row
        oh = oh_b.astype(jnp.bfloat16)
        r = lax.dot_general(ltri, oh, (((1,), (0,)), ((), ())),
                            preferred_element_type=jnp.float32)
        ranks.append(r + run)
        ohs.append(oh_b)
        run = run + jnp.sum(oh.astype(jnp.float32), axis=0, keepdims=True)

    counts = run
    nb = jnp.ceil(counts / BM)
    fr = lax.broadcasted_iota(jnp.int32, (NUM_EXPERTS, NUM_EXPERTS), 0)
    er = lax.broadcasted_iota(jnp.int32, (NUM_EXPERTS, NUM_EXPERTS), 1)
    pre = jnp.where(fr < er, 1.0, 0.0).astype(jnp.bfloat16)
    pstart = lax.dot_general(nb.astype(jnp.bfloat16), pre,
                             (((1,), (0,)), ((), ())),
                             preferred_element_type=jnp.float32)
    pstart_rows = pstart * BM

    for c in range(NCH):
        dvals = pstart_rows + ranks[c]
        d = jnp.sum(jnp.where(ohs[c], dvals, 0.0), axis=1).astype(jnp.int32)
        dest_ref[c, :] = d

    b_col = lax.broadcasted_iota(jnp.int32, (NB, 1), 0).astype(jnp.float32)
    be = jnp.sum((pstart <= b_col).astype(jnp.int32), axis=1) - 1
    be_ref[0, :] = be
    nv_ref[...] = jnp.sum(nb).astype(jnp.int32).reshape(1, 1)


def _route(ids):
    return pl.pallas_call(
        _route_body,
        in_specs=[pl.BlockSpec((M_TOKENS, TOP_K), lambda: (0, 0))],
        out_specs=[
            pl.BlockSpec((NCH, CH), lambda: (0, 0)),
            pl.BlockSpec((1, NB), lambda: (0, 0)),
            pl.BlockSpec((1, 1), lambda: (0, 0)),
        ],
        out_shape=[
            jax.ShapeDtypeStruct((NCH, CH), jnp.int32),
            jax.ShapeDtypeStruct((1, NB), jnp.int32),
            jax.ShapeDtypeStruct((1, 1), jnp.int32),
        ],
    )(ids)


def _sc_mesh():
    return plsc.VectorSubcoreMesh(core_axis_name="c", subcore_axis_name="s")


def _dispatch_sc(x, dest):
    out_t = jax.ShapeDtypeStruct((NROWS, DH), jnp.float32)

    @pl.kernel(out_type=(out_t, out_t), mesh=_sc_mesh())
    def dispatch_kernel(x_hbm, d_hbm, xsl_hbm, xsr_hbm):
        def make_body(target_hbm):
            def body(x_vmem, d_vmem):
                pltpu.sync_copy(x_vmem, target_hbm.at[d_vmem.at[0]])
            return body

        for half, target in ((0, xsl_hbm), (1, xsr_hbm)):
            pltpu.emit_pipeline(
                make_body(target),
                grid=(NPAIRS // W,),
                in_specs=[
                    pl.BlockSpec((W, DH),
                                 index_map=lambda i, h=half: (
                                     i % (M_TOKENS // W), h)),
                    pl.BlockSpec((1, W), index_map=lambda i: (0, i)),
                ],
                out_specs=[],
                core_axis_name=("c", "s"),
                dimension_semantics=(pltpu.PARALLEL,),
            )(x_hbm, d_hbm)

    return dispatch_kernel(x, dest)


def _gather_sc(ysl, ysr, dest):
    out_t = jax.ShapeDtypeStruct((NPAIRS, DH), jnp.float32)

    @pl.kernel(out_type=(out_t, out_t), mesh=_sc_mesh())
    def gather_kernel(ysl_hbm, ysr_hbm, d_hbm, ygl_hbm, ygr_hbm):
        def make_body(src_hbm):
            def body(d_vmem, o_vmem):
                pltpu.sync_copy(src_hbm.at[d_vmem.at[0]], o_vmem)
            return body

        for src, dst in ((ysl_hbm, ygl_hbm), (ysr_hbm, ygr_hbm)):
            pltpu.emit_pipeline(
                make_body(src),
                grid=(NPAIRS // W,),
                in_specs=[pl.BlockSpec((1, W), index_map=lambda i: (0, i))],
                out_specs=[pl.BlockSpec((W, DH),
                                        index_map=lambda i: (i, 0))],
                core_axis_name=("c", "s"),
                dimension_semantics=(pltpu.PARALLEL,),
            )(d_hbm, dst)

    return gather_kernel(ysl, ysr, dest)


def _mm_body(be_ref, nv_ref, xsl_ref, xsr_ref, w1l_ref, w1r_ref, w2_ref,
             ysl_ref, ysr_ref):
    b = pl.program_id(0)

    @pl.when(b < nv_ref[0])
    def _():
        xl = xsl_ref[...].astype(jnp.bfloat16)
        xr = xsr_ref[...].astype(jnp.bfloat16)
        w1l = w1l_ref[0].astype(jnp.bfloat16)
        w1r = w1r_ref[0].astype(jnp.bfloat16)
        h = lax.dot_general(xl, w1l, (((1,), (1,)), ((), ())),
                            preferred_element_type=jnp.float32)
        h += lax.dot_general(xr, w1r, (((1,), (1,)), ((), ())),
                             preferred_element_type=jnp.float32)
        gate = h[:, :D_FF]
        up = h[:, D_FF:]
        act = (jax.nn.sigmoid(gate) * gate * up).astype(jnp.bfloat16)
        w2 = w2_ref[0].astype(jnp.bfloat16)
        y = lax.dot_general(act, w2, (((1,), (1,)), ((), ())),
                            preferred_element_type=jnp.float32)
        ysl_ref[...] = y[:, :DH]
        ysr_ref[...] = y[:, DH:]


def _grouped_mlp(be, nv, xsl, xsr, w1, w2):
    def clamp(b, nv_ref):
        return jnp.minimum(b, nv_ref[0] - 1)

    half_t = jax.ShapeDtypeStruct((NROWS, DH), jnp.float32)
    grid_spec = pltpu.PrefetchScalarGridSpec(
        num_scalar_prefetch=2,
        grid=(NB,),
        in_specs=[
            pl.BlockSpec((BM, DH),
                         lambda b, be_r, nv_r: (clamp(b, nv_r), 0)),
            pl.BlockSpec((BM, DH),
                         lambda b, be_r, nv_r: (clamp(b, nv_r), 0)),
            pl.BlockSpec((1, 2 * D_FF, DH),
                         lambda b, be_r, nv_r: (be_r[clamp(b, nv_r)], 0, 0)),
            pl.BlockSpec((1, 2 * D_FF, DH),
                         lambda b, be_r, nv_r: (be_r[clamp(b, nv_r)], 0, 1)),
            pl.BlockSpec((1, D_MODEL, D_FF),
                         lambda b, be_r, nv_r: (be_r[clamp(b, nv_r)], 0, 0)),
        ],
        out_specs=[
            pl.BlockSpec((BM, DH),
                         lambda b, be_r, nv_r: (clamp(b, nv_r), 0)),
            pl.BlockSpec((BM, DH),
                         lambda b, be_r, nv_r: (clamp(b, nv_r), 0)),
        ],
    )
    return pl.pallas_call(
        _mm_body,
        grid_spec=grid_spec,
        out_shape=[half_t, half_t],
    )(be, nv, xsl, xsr, w1, w1, w2)


def _fin_body(tw_ref, l0_ref, l1_ref, r0_ref, r1_ref, out_ref):
    tw = tw_ref[...]
    w0 = tw[:, 0:1]
    w1 = tw[:, 1:2]
    out_ref[:, :DH] = l0_ref[...] * w0 + l1_ref[...] * w1
    out_ref[:, DH:] = r0_ref[...] * w0 + r1_ref[...] * w1


def _combine(topk_weights, ygl, ygr):
    nblk = M_TOKENS // BT
    return pl.pallas_call(
        _fin_body,
        grid=(nblk,),
        in_specs=[
            pl.BlockSpec((BT, TOP_K), lambda m: (m, 0)),
            pl.BlockSpec((BT, DH), lambda m: (m, 0)),
            pl.BlockSpec((BT, DH), lambda m, n=nblk: (m + n, 0)),
            pl.BlockSpec((BT, DH), lambda m: (m, 0)),
            pl.BlockSpec((BT, DH), lambda m, n=nblk: (m + n, 0)),
        ],
        out_specs=pl.BlockSpec((BT, D_MODEL), lambda m: (m, 0)),
        out_shape=jax.ShapeDtypeStruct((M_TOKENS, D_MODEL), jnp.float32),
    )(topk_weights, ygl, ygl, ygr, ygr)


def kernel(hidden_states, w1, w2, topk_weights, topk_ids):
    ids = topk_ids.astype(jnp.int32)

    dest8, be2, nv2 = _route(ids)
    dest = dest8.reshape(1, NPAIRS)
    be = be2.reshape(NB)
    nv = nv2.reshape(1)

    xsl, xsr = _dispatch_sc(hidden_states, dest)
    ysl, ysr = _grouped_mlp(be, nv, xsl, xsr, w1, w2)
    ygl, ygr = _gather_sc(ysl, ysr, dest)
    return _combine(topk_weights, ygl, ygr)
